# baseline (speedup 1.0000x reference)
import functools

import jax
import jax.numpy as jnp
from jax import lax
from jax.experimental import pallas as pl
from jax.experimental.pallas import tpu as pltpu
from jax.experimental.pallas import tpu_sc as plsc

_CH = 128
_LANES = 16


@jax.jit
def _biased_embedding(index, vect_weight, bias_weight):
    V, D = vect_weight.shape
    (B,) = index.shape
    fold = 128 // D
    vw4 = vect_weight.reshape(V // fold, 128)
    bias_flat = bias_weight.reshape(-1)

    info = plsc.get_sparse_core_info()
    nw = info.num_cores * info.num_subcores
    b_per_w = B // nw
    nch = b_per_w // _CH

    mesh = plsc.VectorSubcoreMesh(core_axis_name="core", subcore_axis_name="subcore")

    scratch = (
        [pltpu.VMEM((_CH,), jnp.int32) for _ in range(nch)]
        + [pltpu.VMEM((_CH,), jnp.int32) for _ in range(nch)]
        + [pltpu.VMEM((_CH, 128), jnp.float32) for _ in range(nch)]
        + [pltpu.VMEM((_CH * D,), jnp.float32) for _ in range(nch)]
        + [pltpu.VMEM((_CH,), jnp.float32) for _ in range(nch)]
        + [pltpu.SemaphoreType.DMA]
    )

    @functools.partial(
        pl.kernel,
        out_type=(
            jax.ShapeDtypeStruct((B,), jnp.float32),
            jax.ShapeDtypeStruct((B * D,), jnp.float32),
        ),
        mesh=mesh,
        scratch_types=scratch,
        compiler_params=pltpu.CompilerParams(
            use_tc_tiling_on_sc=True, needs_layout_passes=False
        ),
    )
    def run(vw4_hbm, bias_hbm, idx_hbm, bias_out, vect_out, *bufs):
        idxb = bufs[:nch]
        srow = bufs[nch:2 * nch]
        win = bufs[2 * nch:3 * nch]
        outw = bufs[3 * nch:4 * nch]
        bbuf = bufs[4 * nch:5 * nch]
        sem = bufs[5 * nch]

        wid = lax.axis_index("subcore") * info.num_cores + lax.axis_index("core")
        base = wid * b_per_w
        lane = lax.iota(jnp.int32, _LANES)

        copies = []
        for w in range(nch):
            wb = base + w * _CH
            pltpu.sync_copy(idx_hbm.at[pl.ds(wb, _CH)], idxb[w])
            for c in range(_CH // _LANES):
                sl = pl.ds(c * _LANES, _LANES)
                srow[w][sl] = idxb[w][sl] >> 2
            copies.append(pltpu.async_copy(vw4_hbm.at[srow[w]], win[w], sem))
            copies.append(pltpu.async_copy(bias_hbm.at[idxb[w]], bbuf[w], sem))
        for c in copies:
            c.wait()
        for w in range(nch):
            for c in range(_CH // _LANES):
                sl = pl.ds(c * _LANES, _LANES)
                krow = lane + c * _LANES
                col0 = (idxb[w][sl] & 3) << 5
                opos = krow << 5
                for d in range(D):
                    vals = plsc.load_gather(win[w], [krow, col0 + d])
                    plsc.store_scatter(outw[w], [opos + d], vals)
            wb = base + w * _CH
            pltpu.sync_copy(outw[w], vect_out.at[pl.ds(wb * D, _CH * D)])
            pltpu.sync_copy(bbuf[w], bias_out.at[pl.ds(wb, _CH)])

    bias, vect_flat = run(vw4, bias_flat, index)
    return bias, vect_flat.reshape(B, D)


def kernel(index, vect_weight, bias_weight):
    return _biased_embedding(index.astype(jnp.int32), vect_weight, bias_weight)


# speedup vs baseline: 1.0499x; 1.0499x over previous
import functools

import jax
import jax.numpy as jnp
from jax import lax
from jax.experimental import pallas as pl
from jax.experimental.pallas import tpu as pltpu
from jax.experimental.pallas import tpu_sc as plsc

_CH = 128


@jax.jit
def _biased_embedding(index, vect_weight, bias_weight):
    V, D = vect_weight.shape
    (B,) = index.shape
    info = plsc.get_sparse_core_info()
    nw = info.num_cores * info.num_subcores
    b_per_w = B // nw
    nch = b_per_w // _CH

    mesh = plsc.VectorSubcoreMesh(core_axis_name="core", subcore_axis_name="subcore")

    scratch = (
        [pltpu.VMEM((_CH,), jnp.int32) for _ in range(nch)]
        + [pltpu.VMEM((_CH, D), jnp.float32) for _ in range(nch)]
        + [pltpu.VMEM((_CH,), jnp.float32) for _ in range(nch)]
        + [pltpu.SemaphoreType.DMA]
    )

    @functools.partial(
        pl.kernel,
        out_type=(
            jax.ShapeDtypeStruct((B,), jnp.float32),
            jax.ShapeDtypeStruct((B, D), jnp.float32),
        ),
        mesh=mesh,
        scratch_types=scratch,
        compiler_params=pltpu.CompilerParams(use_tc_tiling_on_sc=False),
    )
    def run(vect_hbm, bias_hbm, idx_hbm, bias_out, vect_out, *bufs):
        idxb = bufs[:nch]
        vbuf = bufs[nch:2 * nch]
        bbuf = bufs[2 * nch:3 * nch]
        sem = bufs[3 * nch]

        wid = lax.axis_index("subcore") * info.num_cores + lax.axis_index("core")
        base = wid * b_per_w

        copies = []
        for w in range(nch):
            wb = base + w * _CH
            pltpu.sync_copy(idx_hbm.at[pl.ds(wb, _CH)], idxb[w])
            copies.append(pltpu.async_copy(vect_hbm.at[idxb[w]], vbuf[w], sem))
            copies.append(pltpu.async_copy(bias_hbm.at[idxb[w]], bbuf[w], sem))
        for c in copies:
            c.wait()
        for w in range(nch):
            wb = base + w * _CH
            pltpu.sync_copy(vbuf[w], vect_out.at[pl.ds(wb, _CH)])
            pltpu.sync_copy(bbuf[w], bias_out.at[pl.ds(wb, _CH)])

    return run(vect_weight, bias_weight.reshape(-1), index)


def kernel(index, vect_weight, bias_weight):
    return _biased_embedding(index.astype(jnp.int32), vect_weight, bias_weight)
